# baseline (speedup 1.0000x reference)
import functools

import jax
import jax.numpy as jnp
from jax import lax
from jax.experimental import pallas as pl
from jax.experimental.pallas import tpu as pltpu
from jax.experimental.pallas import tpu_sc as plsc

D = 64
LANES = 16
NUM_WORKERS = 32
CHUNK = 1024


def _embed_kernel(idx_hbm, table_hbm, out_hbm, idx_v, rows_v, sem, *, rows_per_worker):
    wid = lax.axis_index("s") * 2 + lax.axis_index("c")
    num_chunks = rows_per_worker // CHUNK

    @pl.loop(0, num_chunks)
    def _chunk(c):
        base = wid * rows_per_worker + c * CHUNK
        pltpu.sync_copy(idx_hbm.at[pl.ds(base, CHUNK)], idx_v)
        pltpu.async_copy(table_hbm.at[idx_v], rows_v, sem).wait()

        @pl.loop(0, CHUNK)
        def _row(r):
            splat = plsc.load_gather(idx_v, [jnp.full((LANES,), r, jnp.int32)])
            scale = jnp.where(splat != 0, jnp.float32(8.0), jnp.float32(0.0))
            for j in range(D // LANES):
                rows_v[r, pl.ds(j * LANES, LANES)] = (
                    rows_v[r, pl.ds(j * LANES, LANES)] * scale
                )

        pltpu.sync_copy(rows_v, out_hbm.at[pl.ds(base, CHUNK)])


def kernel(input_sequence, table):
    B, S = input_sequence.shape
    n = B * S
    assert n % (NUM_WORKERS * CHUNK) == 0
    rows_per_worker = n // NUM_WORKERS
    idx_flat = input_sequence.reshape(n).astype(jnp.int32)

    mesh = plsc.VectorSubcoreMesh(core_axis_name="c", subcore_axis_name="s")
    out = pl.kernel(
        functools.partial(_embed_kernel, rows_per_worker=rows_per_worker),
        out_type=jax.ShapeDtypeStruct((n, D), jnp.float32),
        mesh=mesh,
        compiler_params=pltpu.CompilerParams(
            needs_layout_passes=False, use_tc_tiling_on_sc=False
        ),
        scratch_types=[
            pltpu.VMEM((CHUNK,), jnp.int32),
            pltpu.VMEM((CHUNK, D), jnp.float32),
            pltpu.SemaphoreType.DMA,
        ],
    )(idx_flat, table)
    return out.reshape(B, S, D)
